# baseline (speedup 1.0000x reference)
import jax
import jax.numpy as jnp
from jax import lax
from jax.experimental import pallas as pl
from jax.experimental.pallas import tpu as pltpu
from jax.experimental.pallas import tpu_sc as plsc

VOCAB = 10001
EMBED = 16
SEQ = 200
PRS = 96
CPR = PRS + 8
BATCH = 16384
VP = 10112
NC, NS, L = 2, 16, 16
NW = NC * NS
RPW = BATCH // NW
KB = RPW // 128
WPG = 8
GRID = NW // WPG


def _prep_body(idx_ref, table_ref, w_ref, b_ref, re_ref, tw_ref):
    @pl.when(pl.program_id(0) == 0)
    def _():
        s = (jnp.sum(table_ref[...] * w_ref[...], axis=0) + b_ref[0]) * (1.0 / SEQ)
        tw_ref[...] = jnp.zeros((VP,), jnp.float32)
        tw_ref[pl.ds(0, VOCAB)] = s

    blk = idx_ref[...]
    pa = blk[0:PRS, :] | (blk[PRS:2 * PRS, :] << 16)
    pb = blk[2 * PRS:SEQ, :] | (VOCAB << 16)
    for k in range(WPG * KB):
        re_ref[k * CPR:k * CPR + PRS, :] = pa[:, k * 128:(k + 1) * 128]
        re_ref[k * CPR + PRS:(k + 1) * CPR, :] = pb[:, k * 128:(k + 1) * 128]


def _pool_body(tw_hbm, idx_hbm, out_hbm, tw_v, buf, out_v, sem):
    wid = lax.axis_index("s") * NC + lax.axis_index("c")
    base = wid * (KB * CPR)
    cp = pltpu.async_copy(idx_hbm.at[pl.ds(base, KB * CPR)], buf, sem)
    pltpu.sync_copy(tw_hbm, tw_v)
    cp.wait()
    zero = jnp.zeros((L,), jnp.float32)
    mask16 = jnp.full((L,), 0xFFFF, jnp.int32)

    def chunk(k, carry):
        def step(l, accs):
            out = []
            for g, a in enumerate(accs):
                w = buf[k * CPR + l, pl.ds(g * L, L)]
                a = a + plsc.load_gather(tw_v, [w & mask16])
                a = a + plsc.load_gather(tw_v, [lax.shift_right_logical(w, 16)])
                out.append(a)
            return tuple(out)

        accs = lax.fori_loop(0, CPR, step, (zero,) * 8)
        for g in range(8):
            out_v[pl.ds(k * 128 + g * L, L)] = accs[g]
        return carry

    lax.fori_loop(0, KB, chunk, 0)
    pltpu.sync_copy(out_v, out_hbm.at[pl.ds(wid * RPW, RPW)])


def kernel(inputs, table, dense_w, dense_b):
    idx_t = inputs.astype(jnp.int32).T
    table_t = table.T
    idx_re, tw = pl.pallas_call(
        _prep_body,
        grid=(GRID,),
        in_specs=[
            pl.BlockSpec((SEQ, WPG * RPW), lambda i: (0, i)),
            pl.BlockSpec((EMBED, VOCAB), lambda i: (0, 0)),
            pl.BlockSpec((EMBED, 1), lambda i: (0, 0)),
            pl.BlockSpec((1,), lambda i: (0,)),
        ],
        out_specs=[
            pl.BlockSpec((WPG * KB * CPR, 128), lambda i: (i, 0)),
            pl.BlockSpec((VP,), lambda i: (0,)),
        ],
        out_shape=[
            jax.ShapeDtypeStruct((NW * KB * CPR, 128), jnp.int32),
            jax.ShapeDtypeStruct((VP,), jnp.float32),
        ],
    )(idx_t, table_t, dense_w, dense_b.astype(jnp.float32))

    pool = pl.kernel(
        _pool_body,
        out_type=jax.ShapeDtypeStruct((BATCH,), jnp.float32),
        mesh=plsc.VectorSubcoreMesh(core_axis_name="c", subcore_axis_name="s"),
        scratch_types=[
            pltpu.VMEM((VP,), jnp.float32),
            pltpu.VMEM((KB * CPR, 128), jnp.int32),
            pltpu.VMEM((RPW,), jnp.float32),
            pltpu.SemaphoreType.DMA,
        ],
        compiler_params=pltpu.CompilerParams(needs_layout_passes=False),
    )
    out = pool(tw, idx_re)
    return out.reshape(BATCH, 1)


# speedup vs baseline: 1.0280x; 1.0280x over previous
import jax
import jax.numpy as jnp
from jax import lax
from jax.experimental import pallas as pl
from jax.experimental.pallas import tpu as pltpu
from jax.experimental.pallas import tpu_sc as plsc

VOCAB = 10001
EMBED = 16
SEQ = 200
PRS = 96
CPR = PRS + 8
BATCH = 16384
VP = 10112
NC, NS, L = 2, 16, 16
NW = NC * NS
RPW = BATCH // NW
KB = RPW // 128
WPG = 8
GRID = NW // WPG


def _prep_body(idx_ref, table_ref, w_ref, b_ref, re_ref, tw_ref):
    @pl.when(pl.program_id(0) == 0)
    def _():
        s = (jnp.sum(table_ref[...] * w_ref[...], axis=0) + b_ref[0]) * (1.0 / SEQ)
        tw_ref[...] = jnp.zeros((VP,), jnp.float32)
        tw_ref[pl.ds(0, VOCAB)] = s

    blk = idx_ref[...]
    pa = blk[0:PRS, :] | (blk[PRS:2 * PRS, :] << 16)
    pb = blk[2 * PRS:SEQ, :] | (VOCAB << 16)
    for k in range(WPG * KB):
        re_ref[k * CPR:k * CPR + PRS, :] = pa[:, k * 128:(k + 1) * 128]
        re_ref[k * CPR + PRS:(k + 1) * CPR, :] = pb[:, k * 128:(k + 1) * 128]


def _pool_body(tw_hbm, idx_hbm, out_hbm, tw_v, b0, b1, out_v, s0, s1):
    wid = lax.axis_index("s") * NC + lax.axis_index("c")
    base = wid * (KB * CPR)
    buf, sem = (b0, b1), (s0, s1)

    def start(k):
        return pltpu.async_copy(
            idx_hbm.at[pl.ds(base + k * CPR, CPR)], buf[k % 2], sem[k % 2])

    cps = [None, None]
    cps[0] = start(0)
    pltpu.sync_copy(tw_hbm, tw_v)
    zero = jnp.zeros((L,), jnp.float32)
    mask16 = jnp.full((L,), 0xFFFF, jnp.int32)
    for k in range(KB):
        if k + 1 < KB:
            cps[(k + 1) % 2] = start(k + 1)
        cps[k % 2].wait()
        bk = buf[k % 2]

        def step(l, accs):
            out = []
            for g, a in enumerate(accs):
                w = bk[l, pl.ds(g * L, L)]
                a = a + plsc.load_gather(tw_v, [w & mask16])
                a = a + plsc.load_gather(tw_v, [lax.shift_right_logical(w, 16)])
                out.append(a)
            return tuple(out)

        accs = lax.fori_loop(0, CPR, step, (zero,) * 8)
        for g in range(8):
            out_v[pl.ds(k * 128 + g * L, L)] = accs[g]
    pltpu.sync_copy(out_v, out_hbm.at[pl.ds(wid * RPW, RPW)])


def kernel(inputs, table, dense_w, dense_b):
    idx_t = inputs.astype(jnp.int32).T
    table_t = table.T
    idx_re, tw = pl.pallas_call(
        _prep_body,
        grid=(GRID,),
        in_specs=[
            pl.BlockSpec((SEQ, WPG * RPW), lambda i: (0, i)),
            pl.BlockSpec((EMBED, VOCAB), lambda i: (0, 0)),
            pl.BlockSpec((EMBED, 1), lambda i: (0, 0)),
            pl.BlockSpec((1,), lambda i: (0,)),
        ],
        out_specs=[
            pl.BlockSpec((WPG * KB * CPR, 128), lambda i: (i, 0)),
            pl.BlockSpec((VP,), lambda i: (0,)),
        ],
        out_shape=[
            jax.ShapeDtypeStruct((NW * KB * CPR, 128), jnp.int32),
            jax.ShapeDtypeStruct((VP,), jnp.float32),
        ],
    )(idx_t, table_t, dense_w, dense_b.astype(jnp.float32))

    pool = pl.kernel(
        _pool_body,
        out_type=jax.ShapeDtypeStruct((BATCH,), jnp.float32),
        mesh=plsc.VectorSubcoreMesh(core_axis_name="c", subcore_axis_name="s"),
        scratch_types=[
            pltpu.VMEM((VP,), jnp.float32),
            pltpu.VMEM((CPR, 128), jnp.int32),
            pltpu.VMEM((CPR, 128), jnp.int32),
            pltpu.VMEM((RPW,), jnp.float32),
            pltpu.SemaphoreType.DMA,
            pltpu.SemaphoreType.DMA,
        ],
        compiler_params=pltpu.CompilerParams(needs_layout_passes=False),
    )
    out = pool(tw, idx_re)
    return out.reshape(BATCH, 1)


# speedup vs baseline: 1.0646x; 1.0355x over previous
import jax
import jax.numpy as jnp
from jax import lax
from jax.experimental import pallas as pl
from jax.experimental.pallas import tpu as pltpu
from jax.experimental.pallas import tpu_sc as plsc

VOCAB = 10001
EMBED = 16
SEQ = 200
PRS = 96
CPR = PRS + 8
BATCH = 16384
VP = 10112
NC, NS, L = 2, 16, 16
NW = NC * NS
RPW = BATCH // NW
KB = RPW // 128
WPG = 8
GRID = NW // WPG


def _prep_body(idx_ref, table_ref, w_ref, b_ref, re_ref, tw_ref):
    @pl.when(pl.program_id(0) == 0)
    def _():
        row = jnp.dot(w_ref[...], table_ref[...],
                      preferred_element_type=jnp.float32)
        s = ((row + b_ref[0]) * (1.0 / SEQ)).reshape(VOCAB)
        tw_ref[...] = jnp.zeros((VP,), jnp.float32)
        tw_ref[pl.ds(0, VOCAB)] = s

    blk = idx_ref[...]
    pa = blk[0:PRS, :] | (blk[PRS:2 * PRS, :] << 16)
    pb = blk[2 * PRS:SEQ, :] | (VOCAB << 16)
    for k in range(WPG * KB):
        re_ref[k * CPR:k * CPR + PRS, :] = pa[:, k * 128:(k + 1) * 128]
        re_ref[k * CPR + PRS:(k + 1) * CPR, :] = pb[:, k * 128:(k + 1) * 128]


def _pool_body(tw_hbm, idx_hbm, out_hbm, tw_v, b0, b1, out_v, s0, s1):
    wid = lax.axis_index("s") * NC + lax.axis_index("c")
    base = wid * (KB * CPR)
    buf, sem = (b0, b1), (s0, s1)

    def start(k):
        return pltpu.async_copy(
            idx_hbm.at[pl.ds(base + k * CPR, CPR)], buf[k % 2], sem[k % 2])

    cps = [None, None]
    cps[0] = start(0)
    pltpu.sync_copy(tw_hbm, tw_v)
    zero = jnp.zeros((L,), jnp.float32)
    mask16 = jnp.full((L,), 0xFFFF, jnp.int32)
    for k in range(KB):
        if k + 1 < KB:
            cps[(k + 1) % 2] = start(k + 1)
        cps[k % 2].wait()
        bk = buf[k % 2]

        def step(l, accs):
            out = []
            for g, a in enumerate(accs):
                w = bk[l, pl.ds(g * L, L)]
                a = a + plsc.load_gather(tw_v, [w & mask16])
                a = a + plsc.load_gather(tw_v, [lax.shift_right_logical(w, 16)])
                out.append(a)
            return tuple(out)

        accs = lax.fori_loop(0, CPR, step, (zero,) * 8)
        for g in range(8):
            out_v[pl.ds(k * 128 + g * L, L)] = accs[g]
    pltpu.sync_copy(out_v, out_hbm.at[pl.ds(wid * RPW, RPW)])


def kernel(inputs, table, dense_w, dense_b):
    idx_t = inputs.astype(jnp.int32).T
    table_t = table.T
    idx_re, tw = pl.pallas_call(
        _prep_body,
        grid=(GRID,),
        in_specs=[
            pl.BlockSpec((SEQ, WPG * RPW), lambda i: (0, i)),
            pl.BlockSpec((EMBED, VOCAB), lambda i: (0, 0)),
            pl.BlockSpec((1, EMBED), lambda i: (0, 0)),
            pl.BlockSpec((1,), lambda i: (0,)),
        ],
        out_specs=[
            pl.BlockSpec((WPG * KB * CPR, 128), lambda i: (i, 0)),
            pl.BlockSpec((VP,), lambda i: (0,)),
        ],
        out_shape=[
            jax.ShapeDtypeStruct((NW * KB * CPR, 128), jnp.int32),
            jax.ShapeDtypeStruct((VP,), jnp.float32),
        ],
    )(idx_t, table_t, dense_w.T, dense_b.astype(jnp.float32))

    pool = pl.kernel(
        _pool_body,
        out_type=jax.ShapeDtypeStruct((BATCH,), jnp.float32),
        mesh=plsc.VectorSubcoreMesh(core_axis_name="c", subcore_axis_name="s"),
        scratch_types=[
            pltpu.VMEM((VP,), jnp.float32),
            pltpu.VMEM((CPR, 128), jnp.int32),
            pltpu.VMEM((CPR, 128), jnp.int32),
            pltpu.VMEM((RPW,), jnp.float32),
            pltpu.SemaphoreType.DMA,
            pltpu.SemaphoreType.DMA,
        ],
        compiler_params=pltpu.CompilerParams(needs_layout_passes=False),
    )
    out = pool(tw, idx_re)
    return out.reshape(BATCH, 1)


# speedup vs baseline: 1.0901x; 1.0240x over previous
import jax
import jax.numpy as jnp
from jax import lax
from jax.experimental import pallas as pl
from jax.experimental.pallas import tpu as pltpu
from jax.experimental.pallas import tpu_sc as plsc

VOCAB = 10001
EMBED = 16
SEQ = 200
PRS = 96
CPR = PRS + 8
BATCH = 16384
VP = 10112
NC, NS, L = 2, 16, 16
NW = NC * NS
RPW = BATCH // NW
KB = RPW // 128
WPG = 16
GRID = NW // WPG


def _prep_body(idx_ref, table_ref, w_ref, b_ref, re_ref, tw_ref):
    @pl.when(pl.program_id(0) == 0)
    def _():
        row = jnp.dot(w_ref[...], table_ref[...],
                      preferred_element_type=jnp.float32)
        s = ((row + b_ref[0]) * (1.0 / SEQ)).reshape(VOCAB)
        tw_ref[...] = jnp.zeros((VP,), jnp.float32)
        tw_ref[pl.ds(0, VOCAB)] = s

    blk = idx_ref[...]
    pa = blk[0:PRS, :] | (blk[PRS:2 * PRS, :] << 16)
    pb = blk[2 * PRS:SEQ, :] | (VOCAB << 16)
    for k in range(WPG * KB):
        re_ref[k * CPR:k * CPR + PRS, :] = pa[:, k * 128:(k + 1) * 128]
        re_ref[k * CPR + PRS:(k + 1) * CPR, :] = pb[:, k * 128:(k + 1) * 128]


def _pool_body(tw_hbm, idx_hbm, out_hbm, tw_v, b0, b1, out_v, s0, s1):
    wid = lax.axis_index("s") * NC + lax.axis_index("c")
    base = wid * (KB * CPR)
    buf, sem = (b0, b1), (s0, s1)

    def start(k):
        return pltpu.async_copy(
            idx_hbm.at[pl.ds(base + k * CPR, CPR)], buf[k % 2], sem[k % 2])

    cps = [None, None]
    cps[0] = start(0)
    pltpu.sync_copy(tw_hbm, tw_v)
    zero = jnp.zeros((L,), jnp.float32)
    mask16 = jnp.full((L,), 0xFFFF, jnp.int32)
    for k in range(KB):
        if k + 1 < KB:
            cps[(k + 1) % 2] = start(k + 1)
        cps[k % 2].wait()
        bk = buf[k % 2]

        def step(l, accs):
            out = []
            for g, a in enumerate(accs):
                w = bk[l, pl.ds(g * L, L)]
                a = a + plsc.load_gather(tw_v, [w & mask16])
                a = a + plsc.load_gather(tw_v, [lax.shift_right_logical(w, 16)])
                out.append(a)
            return tuple(out)

        accs = lax.fori_loop(0, CPR, step, (zero,) * 8)
        for g in range(8):
            out_v[pl.ds(k * 128 + g * L, L)] = accs[g]
    pltpu.sync_copy(out_v, out_hbm.at[pl.ds(wid * RPW, RPW)])


def kernel(inputs, table, dense_w, dense_b):
    idx_t = inputs.astype(jnp.int32).T
    table_t = table.T
    idx_re, tw = pl.pallas_call(
        _prep_body,
        grid=(GRID,),
        in_specs=[
            pl.BlockSpec((SEQ, WPG * RPW), lambda i: (0, i)),
            pl.BlockSpec((EMBED, VOCAB), lambda i: (0, 0)),
            pl.BlockSpec((1, EMBED), lambda i: (0, 0)),
            pl.BlockSpec((1,), lambda i: (0,)),
        ],
        out_specs=[
            pl.BlockSpec((WPG * KB * CPR, 128), lambda i: (i, 0)),
            pl.BlockSpec((VP,), lambda i: (0,)),
        ],
        out_shape=[
            jax.ShapeDtypeStruct((NW * KB * CPR, 128), jnp.int32),
            jax.ShapeDtypeStruct((VP,), jnp.float32),
        ],
    )(idx_t, table_t, dense_w.T, dense_b.astype(jnp.float32))

    pool = pl.kernel(
        _pool_body,
        out_type=jax.ShapeDtypeStruct((BATCH,), jnp.float32),
        mesh=plsc.VectorSubcoreMesh(core_axis_name="c", subcore_axis_name="s"),
        scratch_types=[
            pltpu.VMEM((VP,), jnp.float32),
            pltpu.VMEM((CPR, 128), jnp.int32),
            pltpu.VMEM((CPR, 128), jnp.int32),
            pltpu.VMEM((RPW,), jnp.float32),
            pltpu.SemaphoreType.DMA,
            pltpu.SemaphoreType.DMA,
        ],
        compiler_params=pltpu.CompilerParams(needs_layout_passes=False),
    )
    out = pool(tw, idx_re)
    return out.reshape(BATCH, 1)
